# baseline (speedup 1.0000x reference)
import functools

import jax
import jax.numpy as jnp
from jax import lax
from jax.experimental import pallas as pl
from jax.experimental.pallas import tpu as pltpu
from jax.experimental.pallas import tpu_sc as plsc

N_GRAPHS = 1024
NODES_PER_GRAPH = 128
D_FEAT = 128
LANES = 16
NC, NS = 2, 16
NW = NC * NS
GPW = N_GRAPHS // NW
CH = D_FEAT // LANES
SCALE = 1.0 / NODES_PER_GRAPH


NBUF = 6


def _pool_body(feats_hbm, nb_hbm, out_hbm, idx_v, rows_a, rows_b, rows_c,
               rows_d, rows_e, rows_f, out_v, sem_a, sem_b, sem_c, sem_d,
               sem_e, sem_f):
    wid = lax.axis_index("s") * NC + lax.axis_index("c")
    base = wid * GPW
    pltpu.sync_copy(nb_hbm.at[pl.ds(base, GPW)], idx_v)
    bufs = (rows_a, rows_b, rows_c, rows_d, rows_e, rows_f)
    sems = (sem_a, sem_b, sem_c, sem_d, sem_e, sem_f)
    def start(g):
        return pltpu.async_copy(feats_hbm.at[idx_v.at[g]], bufs[g % NBUF],
                                sems[g % NBUF])

    copies = [None] * NBUF
    for g in range(NBUF - 1):
        copies[g] = start(g)
    for g in range(GPW):
        nxt = g + NBUF - 1
        if nxt < GPW:
            copies[nxt % NBUF] = start(nxt)
        copies[g % NBUF].wait()
        rows_v = bufs[g % NBUF]

        def body(r, accs):
            return tuple(accs[c] + rows_v[r, pl.ds(c * LANES, LANES)]
                         for c in range(CH))

        accs = lax.fori_loop(
            0, NODES_PER_GRAPH, body,
            tuple(jnp.zeros((LANES,), jnp.float32) for _ in range(CH)),
            unroll=4)
        for c in range(CH):
            out_v[g, pl.ds(c * LANES, LANES)] = accs[c] * SCALE
    pltpu.sync_copy(out_v, out_hbm.at[pl.ds(base, GPW)])


@jax.jit
def kernel(feats, node_batches):
    mesh = plsc.VectorSubcoreMesh(core_axis_name="c", subcore_axis_name="s")
    f = pl.kernel(
        _pool_body,
        mesh=mesh,
        out_type=jax.ShapeDtypeStruct((N_GRAPHS, D_FEAT), jnp.float32),
        scratch_types=[
            pltpu.VMEM((GPW, NODES_PER_GRAPH), jnp.int32),
        ] + [pltpu.VMEM((NODES_PER_GRAPH, D_FEAT), jnp.float32)] * NBUF + [
            pltpu.VMEM((GPW, D_FEAT), jnp.float32),
        ] + [pltpu.SemaphoreType.DMA] * NBUF,
    )
    return f(feats, node_batches)


# speedup vs baseline: 1.0490x; 1.0490x over previous
import functools

import jax
import jax.numpy as jnp
from jax import lax
from jax.experimental import pallas as pl
from jax.experimental.pallas import tpu as pltpu
from jax.experimental.pallas import tpu_sc as plsc

N_GRAPHS = 1024
NODES_PER_GRAPH = 128
D_FEAT = 128
LANES = 16
NC, NS = 2, 16
NW = NC * NS
GPW = N_GRAPHS // NW
CH = D_FEAT // LANES
SCALE = 1.0 / NODES_PER_GRAPH


NBUF = 6


def _pool_body(feats_hbm, nb_hbm, out_hbm, idx_v, rows_a, rows_b, rows_c,
               rows_d, rows_e, rows_f, out_v, sem_a, sem_b, sem_c, sem_d,
               sem_e, sem_f):
    wid = lax.axis_index("s") * NC + lax.axis_index("c")
    base = wid * GPW
    pltpu.sync_copy(nb_hbm.at[pl.ds(base, GPW)], idx_v)
    bufs = (rows_a, rows_b, rows_c, rows_d, rows_e, rows_f)
    sems = (sem_a, sem_b, sem_c, sem_d, sem_e, sem_f)
    def start(g):
        return pltpu.async_copy(feats_hbm.at[idx_v.at[g]], bufs[g % NBUF],
                                sems[g % NBUF])

    copies = [None] * NBUF
    for g in range(NBUF - 1):
        copies[g] = start(g)
    for g in range(GPW):
        nxt = g + NBUF - 1
        if nxt < GPW:
            copies[nxt % NBUF] = start(nxt)
        copies[g % NBUF].wait()
        rows_v = bufs[g % NBUF]

        def body(r, accs):
            return tuple(accs[c] + rows_v[r, pl.ds(c * LANES, LANES)]
                         for c in range(CH))

        accs = lax.fori_loop(
            0, NODES_PER_GRAPH, body,
            tuple(jnp.zeros((LANES,), jnp.float32) for _ in range(CH)),
            unroll=1)
        for c in range(CH):
            out_v[g, pl.ds(c * LANES, LANES)] = accs[c] * SCALE
    pltpu.sync_copy(out_v, out_hbm.at[pl.ds(base, GPW)])


@jax.jit
def kernel(feats, node_batches):
    mesh = plsc.VectorSubcoreMesh(core_axis_name="c", subcore_axis_name="s")
    f = pl.kernel(
        _pool_body,
        mesh=mesh,
        out_type=jax.ShapeDtypeStruct((N_GRAPHS, D_FEAT), jnp.float32),
        scratch_types=[
            pltpu.VMEM((GPW, NODES_PER_GRAPH), jnp.int32),
        ] + [pltpu.VMEM((NODES_PER_GRAPH, D_FEAT), jnp.float32)] * NBUF + [
            pltpu.VMEM((GPW, D_FEAT), jnp.float32),
        ] + [pltpu.SemaphoreType.DMA] * NBUF,
    )
    return f(feats, node_batches)


# speedup vs baseline: 1.0749x; 1.0247x over previous
import functools

import jax
import jax.numpy as jnp
from jax import lax
from jax.experimental import pallas as pl
from jax.experimental.pallas import tpu as pltpu
from jax.experimental.pallas import tpu_sc as plsc

N_GRAPHS = 1024
NODES_PER_GRAPH = 128
D_FEAT = 128
LANES = 16
NC, NS = 2, 16
NW = NC * NS
GPW = N_GRAPHS // NW
CH = D_FEAT // LANES
SCALE = 1.0 / NODES_PER_GRAPH


NBUF = 6


def _pool_body(feats_hbm, nb_hbm, out_hbm, idx_v, rows_a, rows_b, rows_c,
               rows_d, rows_e, rows_f, out_v, sem_a, sem_b, sem_c, sem_d,
               sem_e, sem_f):
    wid = lax.axis_index("s") * NC + lax.axis_index("c")
    base = wid * GPW
    pltpu.sync_copy(nb_hbm.at[pl.ds(base, GPW)], idx_v)
    bufs = (rows_a, rows_b, rows_c, rows_d, rows_e, rows_f)
    sems = (sem_a, sem_b, sem_c, sem_d, sem_e, sem_f)
    for b in range(NBUF):
        pltpu.async_copy(feats_hbm.at[idx_v.at[b]], bufs[b], sems[b])

    def super_body(t, carry):
        for b in range(NBUF):
            g = t * NBUF + b
            rows_v = bufs[b]
            pltpu.make_async_copy(
                feats_hbm.at[pl.ds(0, NODES_PER_GRAPH)], rows_v,
                sems[b]).wait()

            def body(r, accs):
                return tuple(accs[c] + rows_v[r, pl.ds(c * LANES, LANES)]
                             for c in range(CH))

            accs = lax.fori_loop(
                0, NODES_PER_GRAPH, body,
                tuple(jnp.zeros((LANES,), jnp.float32) for _ in range(CH)),
                unroll=1)
            for c in range(CH):
                out_v[g, pl.ds(c * LANES, LANES)] = accs[c] * SCALE
            gn = g + NBUF

            @pl.when(gn < GPW)
            def _():
                pltpu.async_copy(feats_hbm.at[idx_v.at[gn]], rows_v, sems[b])

        return carry

    lax.fori_loop(0, GPW // NBUF, super_body, 0)
    pltpu.sync_copy(out_v, out_hbm.at[pl.ds(base, GPW)])


@jax.jit
def kernel(feats, node_batches):
    mesh = plsc.VectorSubcoreMesh(core_axis_name="c", subcore_axis_name="s")
    f = pl.kernel(
        _pool_body,
        mesh=mesh,
        out_type=jax.ShapeDtypeStruct((N_GRAPHS, D_FEAT), jnp.float32),
        scratch_types=[
            pltpu.VMEM((GPW, NODES_PER_GRAPH), jnp.int32),
        ] + [pltpu.VMEM((NODES_PER_GRAPH, D_FEAT), jnp.float32)] * NBUF + [
            pltpu.VMEM((GPW, D_FEAT), jnp.float32),
        ] + [pltpu.SemaphoreType.DMA] * NBUF,
    )
    return f(feats, node_batches)


# speedup vs baseline: 1.1123x; 1.0348x over previous
import functools

import jax
import jax.numpy as jnp
from jax import lax
from jax.experimental import pallas as pl
from jax.experimental.pallas import tpu as pltpu
from jax.experimental.pallas import tpu_sc as plsc

N_GRAPHS = 1024
NODES_PER_GRAPH = 128
D_FEAT = 128
LANES = 16
NC, NS = 2, 16
NW = NC * NS
GPW = N_GRAPHS // NW
CH = D_FEAT // LANES
SCALE = 1.0 / NODES_PER_GRAPH


NBUF = 4


def _pool_body(feats_hbm, nb_hbm, out_hbm, idx_v, rows_a, rows_b, rows_c,
               rows_d, out_v, sem_a, sem_b, sem_c, sem_d):
    wid = lax.axis_index("s") * NC + lax.axis_index("c")
    base = wid * GPW
    pltpu.sync_copy(nb_hbm.at[pl.ds(base, GPW)], idx_v)
    bufs = (rows_a, rows_b, rows_c, rows_d)
    sems = (sem_a, sem_b, sem_c, sem_d)
    for b in range(NBUF):
        pltpu.async_copy(feats_hbm.at[idx_v.at[b]], bufs[b], sems[b])

    def super_body(t, carry):
        for b in range(NBUF):
            g = t * NBUF + b
            rows_v = bufs[b]
            pltpu.make_async_copy(
                feats_hbm.at[pl.ds(0, NODES_PER_GRAPH)], rows_v,
                sems[b]).wait()

            def body(r, accs):
                return tuple(accs[c] + rows_v[r, pl.ds(c * LANES, LANES)]
                             for c in range(CH))

            accs = lax.fori_loop(
                0, NODES_PER_GRAPH, body,
                tuple(jnp.zeros((LANES,), jnp.float32) for _ in range(CH)),
                unroll=1)
            for c in range(CH):
                out_v[g, pl.ds(c * LANES, LANES)] = accs[c] * SCALE
            gn = g + NBUF

            @pl.when(gn < GPW)
            def _():
                pltpu.async_copy(feats_hbm.at[idx_v.at[gn]], rows_v, sems[b])

        return carry

    lax.fori_loop(0, GPW // NBUF, super_body, 0)
    pltpu.sync_copy(out_v, out_hbm.at[pl.ds(base, GPW)])


@jax.jit
def kernel(feats, node_batches):
    mesh = plsc.VectorSubcoreMesh(core_axis_name="c", subcore_axis_name="s")
    f = pl.kernel(
        _pool_body,
        mesh=mesh,
        out_type=jax.ShapeDtypeStruct((N_GRAPHS, D_FEAT), jnp.float32),
        scratch_types=[
            pltpu.VMEM((GPW, NODES_PER_GRAPH), jnp.int32),
        ] + [pltpu.VMEM((NODES_PER_GRAPH, D_FEAT), jnp.float32)] * NBUF + [
            pltpu.VMEM((GPW, D_FEAT), jnp.float32),
        ] + [pltpu.SemaphoreType.DMA] * NBUF,
    )
    return f(feats, node_batches)
